# Initial kernel scaffold; baseline (speedup 1.0000x reference)
import jax
import jax.numpy as jnp
from jax.experimental import pallas as pl


def kernel(pts, W_head, Wb, bb, Wf):
    raise NotImplementedError("write your pallas kernel here")


# speedup vs baseline: 1.1863x; 1.1863x over previous
import jax
import jax.numpy as jnp
import numpy as np
from jax.experimental import pallas as pl
from jax.experimental.pallas import tpu as pltpu

BN_INV = float(1.0 / np.sqrt(1.0 + 1e-5))
K = 16
N_BLOCKS = 14


def _pairwise_sqdist(x):
    x2 = jnp.sum(x * x, axis=-1)
    return x2[:, :, None] + x2[:, None, :] - 2.0 * jnp.einsum('bnc,bmc->bnm', x, x)


def _knn_idx(x, k):
    d = jax.lax.stop_gradient(_pairwise_sqdist(x))
    _, idx = jax.lax.top_k(-d, k)
    return idx


def _gather_neighbors(x, idx):
    return jax.vmap(lambda xb, ib: xb[ib])(x, idx)


def _edge_conv(x, idx, W, b):
    x_j = _gather_neighbors(x, idx)
    x_i = jnp.broadcast_to(x[:, :, None, :], x_j.shape)
    msg = jnp.concatenate([x_i, x_j - x_i], axis=-1)
    h = jnp.einsum('bnkc,cd->bnkd', msg, W)
    if b is not None:
        h = h + b
    h = jax.nn.relu(h * BN_INV)
    return jnp.max(h, axis=2)


def _fusion_kernel(f_ref, wf_ref, o_ref):
    acc = jnp.dot(f_ref[...], wf_ref[...], preferred_element_type=jnp.float32)
    acc = acc * BN_INV
    o_ref[...] = jnp.where(acc >= 0.0, acc, 0.2 * acc)


def _fusion(f, Wf):
    B, N, C = f.shape
    E = Wf.shape[1]
    out = pl.pallas_call(
        _fusion_kernel,
        grid=(B,),
        in_specs=[
            pl.BlockSpec((1, N, C), lambda b: (b, 0, 0)),
            pl.BlockSpec((C, E), lambda b: (0, 0)),
        ],
        out_specs=pl.BlockSpec((1, N, E), lambda b: (b, 0, 0)),
        out_shape=jax.ShapeDtypeStruct((B, N, E), jnp.float32),
    )(f, Wf)
    return out


def kernel(pts, W_head, Wb, bb, Wf):
    idx0 = _knn_idx(pts, K)
    feats = [_edge_conv(pts, idx0, W_head, None)]
    for i in range(N_BLOCKS - 1):
        dil = i + 1
        xi = feats[-1]
        idx = _knn_idx(xi, K * dil)[:, :, ::dil]
        feats.append(_edge_conv(xi, idx, Wb[i], bb[i]) + xi)
    f = jnp.concatenate(feats, axis=-1)
    out = _fusion(f, Wf)
    return jnp.transpose(out, (0, 2, 1))


# speedup vs baseline: 1.4550x; 1.2265x over previous
import functools

import jax
import jax.numpy as jnp
import numpy as np
from jax import lax
from jax.experimental import pallas as pl
from jax.experimental.pallas import tpu as pltpu

BN_INV = float(1.0 / np.sqrt(1.0 + 1e-5))
K = 16
N_BLOCKS = 14
N = 1024


def _ce(ka, ia, kb, ib, asc):
    swap = (ka > kb) | ((ka == kb) & (ia > ib))
    if not asc:
        swap = ~swap
    lo_k = jnp.where(swap, kb, ka)
    hi_k = jnp.where(swap, ka, kb)
    lo_i = jnp.where(swap, ib, ia)
    hi_i = jnp.where(swap, ia, ib)
    return lo_k, lo_i, hi_k, hi_i


def _bitonic_pass(Kv, Iv, sigma, t):
    E = 1 << t
    if sigma >= 10:
        shape = (1, 1, N >> (1 + t), 2, E, 8, 128)
    else:
        shape = (1 << (9 - sigma), 2, 1 << (sigma - 1 - t), 2, E, 8, 128)
    Kv = Kv.reshape(shape)
    Iv = Iv.reshape(shape)
    parts_k, parts_i = [], []
    for d in range(shape[1]):
        asc = (d == 0)
        lo_k, lo_i, hi_k, hi_i = _ce(Kv[:, d, :, 0], Iv[:, d, :, 0],
                                     Kv[:, d, :, 1], Iv[:, d, :, 1], asc)
        parts_k.append(jnp.stack([lo_k, hi_k], axis=2))
        parts_i.append(jnp.stack([lo_i, hi_i], axis=2))
    Kn = parts_k[0] if len(parts_k) == 1 else jnp.stack(parts_k, axis=1)
    In = parts_i[0] if len(parts_i) == 1 else jnp.stack(parts_i, axis=1)
    return Kn.reshape(N, 8, 128), In.reshape(N, 8, 128)


def _knn_kernel(x_ref, x2_ref, idx_ref, *, dil):
    x = x_ref[0]
    x2 = x2_ref[0, 0]
    g = lax.dot_general(x, x, (((1,), (1,)), ((), ())),
                        preferred_element_type=jnp.float32)
    d = x2[:, None] + x2[None, :] - 2.0 * g
    Kv = d.reshape(N, 8, 128)
    Iv = lax.broadcasted_iota(jnp.int32, (N, 8, 128), 0)
    for sigma in range(1, 11):
        for t in range(sigma - 1, -1, -1):
            Kv, Iv = _bitonic_pass(Kv, Iv, sigma, t)
    sel = jnp.concatenate([Iv[t * dil:t * dil + 1] for t in range(K)], axis=0)
    idx_ref[0] = sel.reshape(K, N)


def _knn_strided(x, dil):
    B = x.shape[0]
    C = x.shape[2]
    x2 = jnp.sum(x * x, axis=-1).reshape(B, 1, N)
    idx16 = pl.pallas_call(
        functools.partial(_knn_kernel, dil=dil),
        grid=(B,),
        in_specs=[
            pl.BlockSpec((1, N, C), lambda b: (b, 0, 0)),
            pl.BlockSpec((1, 1, N), lambda b: (b, 0, 0)),
        ],
        out_specs=pl.BlockSpec((1, K, N), lambda b: (b, 0, 0)),
        out_shape=jax.ShapeDtypeStruct((B, K, N), jnp.int32),
    )(x, x2)
    return jnp.transpose(idx16, (0, 2, 1))


def _gather_neighbors(x, idx):
    return jax.vmap(lambda xb, ib: xb[ib])(x, idx)


def _edge_conv(x, idx, W, b):
    x_j = _gather_neighbors(x, idx)
    x_i = jnp.broadcast_to(x[:, :, None, :], x_j.shape)
    msg = jnp.concatenate([x_i, x_j - x_i], axis=-1)
    h = jnp.einsum('bnkc,cd->bnkd', msg, W)
    if b is not None:
        h = h + b
    h = jax.nn.relu(h * BN_INV)
    return jnp.max(h, axis=2)


def _fusion_kernel(f_ref, wf_ref, o_ref):
    acc = jnp.dot(f_ref[...], wf_ref[...], preferred_element_type=jnp.float32)
    acc = acc * BN_INV
    o_ref[...] = jnp.where(acc >= 0.0, acc, 0.2 * acc)


def _fusion(f, Wf):
    B, n, C = f.shape
    E = Wf.shape[1]
    return pl.pallas_call(
        _fusion_kernel,
        grid=(B,),
        in_specs=[
            pl.BlockSpec((1, n, C), lambda b: (b, 0, 0)),
            pl.BlockSpec((C, E), lambda b: (0, 0)),
        ],
        out_specs=pl.BlockSpec((1, n, E), lambda b: (b, 0, 0)),
        out_shape=jax.ShapeDtypeStruct((B, n, E), jnp.float32),
    )(f, Wf)


def kernel(pts, W_head, Wb, bb, Wf):
    idx0 = _knn_strided(pts, 1)
    feats = [_edge_conv(pts, idx0, W_head, None)]
    for i in range(N_BLOCKS - 1):
        dil = i + 1
        xi = feats[-1]
        idx = _knn_strided(xi, dil)
        feats.append(_edge_conv(xi, idx, Wb[i], bb[i]) + xi)
    f = jnp.concatenate(feats, axis=-1)
    out = _fusion(f, Wf)
    return jnp.transpose(out, (0, 2, 1))


# speedup vs baseline: 4.2902x; 2.9486x over previous
import functools

import jax
import jax.numpy as jnp
import numpy as np
from jax import lax
from jax.experimental import pallas as pl
from jax.experimental.pallas import tpu as pltpu
from jax.experimental.pallas import tpu_sc as plsc

BN_INV = float(1.0 / np.sqrt(1.0 + 1e-5))
K = 16
N_BLOCKS = 14
N = 1024


def _ce(ka, ia, kb, ib, asc):
    swap = (ka > kb) | ((ka == kb) & (ia > ib))
    if not asc:
        swap = ~swap
    lo_k = jnp.where(swap, kb, ka)
    hi_k = jnp.where(swap, ka, kb)
    lo_i = jnp.where(swap, ib, ia)
    hi_i = jnp.where(swap, ia, ib)
    return lo_k, lo_i, hi_k, hi_i


def _bitonic_pass(Kv, Iv, sigma, t):
    E = 1 << t
    if sigma >= 10:
        shape = (1, 1, N >> (1 + t), 2, E, 8, 128)
    else:
        shape = (1 << (9 - sigma), 2, 1 << (sigma - 1 - t), 2, E, 8, 128)
    Kv = Kv.reshape(shape)
    Iv = Iv.reshape(shape)
    parts_k, parts_i = [], []
    for d in range(shape[1]):
        asc = (d == 0)
        lo_k, lo_i, hi_k, hi_i = _ce(Kv[:, d, :, 0], Iv[:, d, :, 0],
                                     Kv[:, d, :, 1], Iv[:, d, :, 1], asc)
        parts_k.append(jnp.stack([lo_k, hi_k], axis=2))
        parts_i.append(jnp.stack([lo_i, hi_i], axis=2))
    Kn = parts_k[0] if len(parts_k) == 1 else jnp.stack(parts_k, axis=1)
    In = parts_i[0] if len(parts_i) == 1 else jnp.stack(parts_i, axis=1)
    return Kn.reshape(N, 8, 128), In.reshape(N, 8, 128)


def _knn_kernel(x_ref, x2_ref, idx_ref, *, dil):
    x = x_ref[0]
    x2 = x2_ref[0, 0]
    g = lax.dot_general(x, x, (((1,), (1,)), ((), ())),
                        preferred_element_type=jnp.float32)
    d = x2[:, None] + x2[None, :] - 2.0 * g
    Kv = d.reshape(N, 8, 128)
    Iv = lax.broadcasted_iota(jnp.int32, (N, 8, 128), 0)
    for sigma in range(1, 11):
        for t in range(sigma - 1, -1, -1):
            Kv, Iv = _bitonic_pass(Kv, Iv, sigma, t)
    sel = jnp.concatenate([Iv[t * dil:t * dil + 1] for t in range(K)], axis=0)
    idx_ref[0] = sel.reshape(K, N)


def _knn_strided(x, dil):
    B = x.shape[0]
    C = x.shape[2]
    x2 = jnp.sum(x * x, axis=-1).reshape(B, 1, N)
    idx16 = pl.pallas_call(
        functools.partial(_knn_kernel, dil=dil),
        grid=(B,),
        in_specs=[
            pl.BlockSpec((1, N, C), lambda b: (b, 0, 0)),
            pl.BlockSpec((1, 1, N), lambda b: (b, 0, 0)),
        ],
        out_specs=pl.BlockSpec((1, K, N), lambda b: (b, 0, 0)),
        out_shape=jax.ShapeDtypeStruct((B, K, N), jnp.int32),
    )(x, x2)
    return jnp.transpose(idx16, (0, 2, 1))


_GCHUNK = 512


def _make_sc_gather(total_rows, n_idx, D):
    mesh = plsc.VectorSubcoreMesh(core_axis_name="c", subcore_axis_name="s")
    per_w = n_idx // 32
    n_chunks = per_w // _GCHUNK

    @functools.partial(
        pl.kernel,
        mesh=mesh,
        out_type=jax.ShapeDtypeStruct((n_idx, D), jnp.float32),
        scratch_types=[
            pltpu.VMEM((_GCHUNK,), jnp.int32),
            pltpu.VMEM((_GCHUNK, D), jnp.float32),
            pltpu.SemaphoreType.DMA,
        ],
    )
    def gath(table_hbm, idx_hbm, out_hbm, idx_v, rows_v, sem):
        wid = lax.axis_index("s") * 2 + lax.axis_index("c")
        base = wid * per_w
        for ci in range(n_chunks):
            off = base + ci * _GCHUNK
            pltpu.sync_copy(idx_hbm.at[pl.ds(off, _GCHUNK)], idx_v)
            pltpu.async_copy(table_hbm.at[idx_v], rows_v, sem).wait()
            pltpu.sync_copy(rows_v, out_hbm.at[pl.ds(off, _GCHUNK)])

    return gath


def _gather_neighbors(x, idx):
    B, n, C = x.shape
    Dp = 128
    xt = x.reshape(B * n, C)
    if Dp != C:
        xt = jnp.pad(xt, ((0, 0), (0, Dp - C)))
    gidx = (idx + (jnp.arange(B, dtype=idx.dtype) * n)[:, None, None]).reshape(-1)
    rows = _make_sc_gather(B * n, B * n * K, Dp)(xt, gidx)
    return rows.reshape(B, n, K, Dp)[..., :C]


def _edge_conv(x, idx, W, b):
    x_j = _gather_neighbors(x, idx)
    x_i = jnp.broadcast_to(x[:, :, None, :], x_j.shape)
    msg = jnp.concatenate([x_i, x_j - x_i], axis=-1)
    h = jnp.einsum('bnkc,cd->bnkd', msg, W)
    if b is not None:
        h = h + b
    h = jax.nn.relu(h * BN_INV)
    return jnp.max(h, axis=2)


def _fusion_kernel(f_ref, wf_ref, o_ref):
    acc = jnp.dot(f_ref[...], wf_ref[...], preferred_element_type=jnp.float32)
    acc = acc * BN_INV
    o_ref[...] = jnp.where(acc >= 0.0, acc, 0.2 * acc)


def _fusion(f, Wf):
    B, n, C = f.shape
    E = Wf.shape[1]
    return pl.pallas_call(
        _fusion_kernel,
        grid=(B,),
        in_specs=[
            pl.BlockSpec((1, n, C), lambda b: (b, 0, 0)),
            pl.BlockSpec((C, E), lambda b: (0, 0)),
        ],
        out_specs=pl.BlockSpec((1, n, E), lambda b: (b, 0, 0)),
        out_shape=jax.ShapeDtypeStruct((B, n, E), jnp.float32),
    )(f, Wf)


def kernel(pts, W_head, Wb, bb, Wf):
    idx0 = _knn_strided(pts, 1)
    feats = [_edge_conv(pts, idx0, W_head, None)]
    for i in range(N_BLOCKS - 1):
        dil = i + 1
        xi = feats[-1]
        idx = _knn_strided(xi, dil)
        feats.append(_edge_conv(xi, idx, Wb[i], bb[i]) + xi)
    f = jnp.concatenate(feats, axis=-1)
    out = _fusion(f, Wf)
    return jnp.transpose(out, (0, 2, 1))


# speedup vs baseline: 4.6374x; 1.0809x over previous
import functools

import jax
import jax.numpy as jnp
import numpy as np
from jax import lax
from jax.experimental import pallas as pl
from jax.experimental.pallas import tpu as pltpu
from jax.experimental.pallas import tpu_sc as plsc

BN_INV = float(1.0 / np.sqrt(1.0 + 1e-5))
K = 16
N_BLOCKS = 14
N = 1024


def _ce(ka, ia, kb, ib, asc):
    swap = (ka > kb) | ((ka == kb) & (ia > ib))
    if not asc:
        swap = ~swap
    lo_k = jnp.where(swap, kb, ka)
    hi_k = jnp.where(swap, ka, kb)
    lo_i = jnp.where(swap, ib, ia)
    hi_i = jnp.where(swap, ia, ib)
    return lo_k, lo_i, hi_k, hi_i


def _bitonic_pass(Kv, Iv, sigma, t):
    E = 1 << t
    if sigma >= 10:
        shape = (1, 1, N >> (1 + t), 2, E, 8, 128)
    else:
        shape = (1 << (9 - sigma), 2, 1 << (sigma - 1 - t), 2, E, 8, 128)
    Kv = Kv.reshape(shape)
    Iv = Iv.reshape(shape)
    parts_k, parts_i = [], []
    for d in range(shape[1]):
        asc = (d == 0)
        lo_k, lo_i, hi_k, hi_i = _ce(Kv[:, d, :, 0], Iv[:, d, :, 0],
                                     Kv[:, d, :, 1], Iv[:, d, :, 1], asc)
        parts_k.append(jnp.stack([lo_k, hi_k], axis=2))
        parts_i.append(jnp.stack([lo_i, hi_i], axis=2))
    Kn = parts_k[0] if len(parts_k) == 1 else jnp.stack(parts_k, axis=1)
    In = parts_i[0] if len(parts_i) == 1 else jnp.stack(parts_i, axis=1)
    return Kn.reshape(N, 8, 128), In.reshape(N, 8, 128)


def _knn_kernel(x_ref, x2_ref, idx_ref, *, dil):
    x = x_ref[0]
    x2 = x2_ref[0, 0]
    g = lax.dot_general(x, x, (((1,), (1,)), ((), ())),
                        preferred_element_type=jnp.float32)
    d = x2[:, None] + x2[None, :] - 2.0 * g
    Kv = d.reshape(N, 8, 128)
    Iv = lax.broadcasted_iota(jnp.int32, (N, 8, 128), 0)
    for sigma in range(1, 11):
        for t in range(sigma - 1, -1, -1):
            Kv, Iv = _bitonic_pass(Kv, Iv, sigma, t)
    sel = jnp.concatenate([Iv[t * dil:t * dil + 1] for t in range(K)], axis=0)
    idx_ref[0] = sel.reshape(K, N)


def _knn_strided(x, dil):
    B = x.shape[0]
    C = x.shape[2]
    x2 = jnp.sum(x * x, axis=-1).reshape(B, 1, N)
    idx16 = pl.pallas_call(
        functools.partial(_knn_kernel, dil=dil),
        grid=(B,),
        in_specs=[
            pl.BlockSpec((1, N, C), lambda b: (b, 0, 0)),
            pl.BlockSpec((1, 1, N), lambda b: (b, 0, 0)),
        ],
        out_specs=pl.BlockSpec((1, K, N), lambda b: (b, 0, 0)),
        out_shape=jax.ShapeDtypeStruct((B, K, N), jnp.int32),
    )(x, x2)
    return jnp.transpose(idx16, (0, 2, 1))


_GCHUNK = 512


def _make_sc_gather(total_rows, n_idx, D):
    mesh = plsc.VectorSubcoreMesh(core_axis_name="c", subcore_axis_name="s")
    per_w = n_idx // 32
    n_chunks = per_w // _GCHUNK

    @functools.partial(
        pl.kernel,
        mesh=mesh,
        out_type=jax.ShapeDtypeStruct((n_idx, D), jnp.float32),
        scratch_types=[
            pltpu.VMEM((_GCHUNK,), jnp.int32),
            pltpu.VMEM((_GCHUNK, D), jnp.float32),
            pltpu.SemaphoreType.DMA,
        ],
    )
    def gath(table_hbm, idx_hbm, out_hbm, idx_v, rows_v, sem):
        wid = lax.axis_index("s") * 2 + lax.axis_index("c")
        base = wid * per_w
        for ci in range(n_chunks):
            off = base + ci * _GCHUNK
            pltpu.sync_copy(idx_hbm.at[pl.ds(off, _GCHUNK)], idx_v)
            pltpu.async_copy(table_hbm.at[idx_v], rows_v, sem).wait()
            pltpu.sync_copy(rows_v, out_hbm.at[pl.ds(off, _GCHUNK)])

    return gath


def _gather_neighbors(x, idx):
    B, n, C = x.shape
    Dp = 128
    xt = x.reshape(B * n, C)
    if Dp != C:
        xt = jnp.pad(xt, ((0, 0), (0, Dp - C)))
    gidx = (idx + (jnp.arange(B, dtype=idx.dtype) * n)[:, None, None]).reshape(-1)
    return _make_sc_gather(B * n, B * n * K, Dp)(xt, gidx)


_ECHUNK = 256


def _edge_conv_kernel(x_ref, xj_ref, w_ref, b_ref, o_ref, *, C, residual):
    xi = x_ref[0]
    xj = xj_ref[0][:, :C]
    xib = jnp.broadcast_to(xi[:, None, :], (_ECHUNK, K, C)).reshape(_ECHUNK * K, C)
    msg = jnp.concatenate([xib, xj - xib], axis=-1)
    h = lax.dot_general(msg, w_ref[...], (((1,), (0,)), ((), ())),
                        preferred_element_type=jnp.float32)
    if b_ref is not None:
        h = h + b_ref[0]
    h = jnp.maximum(h * BN_INV, 0.0)
    out = jnp.max(h.reshape(_ECHUNK, K, h.shape[-1]), axis=1)
    if residual:
        out = out + xi
    o_ref[0] = out


def _edge_conv(x, xj_rows, W, b, residual):
    B, n, C = x.shape
    Cout = W.shape[1]
    nchunks = n // _ECHUNK
    xj3 = xj_rows.reshape(B, n * K, 128)
    args = [x, xj3, W]
    in_specs = [
        pl.BlockSpec((1, _ECHUNK, C), lambda b_, c_: (b_, c_, 0)),
        pl.BlockSpec((1, _ECHUNK * K, 128), lambda b_, c_: (b_, c_, 0)),
        pl.BlockSpec(W.shape, lambda b_, c_: (0, 0)),
    ]
    if b is not None:
        args.append(b.reshape(1, Cout))
        in_specs.append(pl.BlockSpec((1, Cout), lambda b_, c_: (0, 0)))
        body = functools.partial(_edge_conv_kernel, C=C, residual=residual)
    else:
        body = functools.partial(
            lambda x_ref, xj_ref, w_ref, o_ref, C, residual:
            _edge_conv_kernel(x_ref, xj_ref, w_ref, None, o_ref,
                              C=C, residual=residual),
            C=C, residual=residual)
    return pl.pallas_call(
        body,
        grid=(B, nchunks),
        in_specs=in_specs,
        out_specs=pl.BlockSpec((1, _ECHUNK, Cout), lambda b_, c_: (b_, c_, 0)),
        out_shape=jax.ShapeDtypeStruct((B, n, Cout), jnp.float32),
    )(*args)


def _fusion_kernel(f_ref, wf_ref, o_ref):
    acc = jnp.dot(f_ref[...], wf_ref[...], preferred_element_type=jnp.float32)
    acc = acc * BN_INV
    o_ref[...] = jnp.where(acc >= 0.0, acc, 0.2 * acc)


def _fusion(f, Wf):
    B, n, C = f.shape
    E = Wf.shape[1]
    return pl.pallas_call(
        _fusion_kernel,
        grid=(B,),
        in_specs=[
            pl.BlockSpec((1, n, C), lambda b: (b, 0, 0)),
            pl.BlockSpec((C, E), lambda b: (0, 0)),
        ],
        out_specs=pl.BlockSpec((1, n, E), lambda b: (b, 0, 0)),
        out_shape=jax.ShapeDtypeStruct((B, n, E), jnp.float32),
    )(f, Wf)


def kernel(pts, W_head, Wb, bb, Wf):
    idx0 = _knn_strided(pts, 1)
    xj0 = _gather_neighbors(pts, idx0)
    feats = [_edge_conv(pts, xj0, W_head, None, residual=False)]
    for i in range(N_BLOCKS - 1):
        dil = i + 1
        xi = feats[-1]
        idx = _knn_strided(xi, dil)
        xj = _gather_neighbors(xi, idx)
        feats.append(_edge_conv(xi, xj, Wb[i], bb[i], residual=True))
    f = jnp.concatenate(feats, axis=-1)
    out = _fusion(f, Wf)
    return jnp.transpose(out, (0, 2, 1))


# speedup vs baseline: 6.8668x; 1.4807x over previous
import functools

import jax
import jax.numpy as jnp
import numpy as np
from jax import lax
from jax.experimental import pallas as pl
from jax.experimental.pallas import tpu as pltpu
from jax.experimental.pallas import tpu_sc as plsc

BN_INV = float(1.0 / np.sqrt(1.0 + 1e-5))
K = 16
N_BLOCKS = 14
N = 1024


def _ce(ka, ia, kb, ib, asc):
    swap = (ka > kb) | ((ka == kb) & (ia > ib))
    if not asc:
        swap = ~swap
    lo_k = jnp.where(swap, kb, ka)
    hi_k = jnp.where(swap, ka, kb)
    lo_i = jnp.where(swap, ib, ia)
    hi_i = jnp.where(swap, ia, ib)
    return lo_k, lo_i, hi_k, hi_i


def _bitonic_pass(Kv, Iv, sigma, t, L):
    S = 1 << L
    E = 1 << t
    if sigma >= L:
        shape = (1, 1, S >> (1 + t), 2, E, 8, 128)
    else:
        shape = (1 << (L - 1 - sigma), 2, 1 << (sigma - 1 - t), 2, E, 8, 128)
    Kv = Kv.reshape(shape)
    Iv = Iv.reshape(shape)
    parts_k, parts_i = [], []
    for d in range(shape[1]):
        asc = (d == 0)
        lo_k, lo_i, hi_k, hi_i = _ce(Kv[:, d, :, 0], Iv[:, d, :, 0],
                                     Kv[:, d, :, 1], Iv[:, d, :, 1], asc)
        parts_k.append(jnp.stack([lo_k, hi_k], axis=2))
        parts_i.append(jnp.stack([lo_i, hi_i], axis=2))
    Kn = parts_k[0] if len(parts_k) == 1 else jnp.stack(parts_k, axis=1)
    In = parts_i[0] if len(parts_i) == 1 else jnp.stack(parts_i, axis=1)
    return Kn.reshape(S, 8, 128), In.reshape(S, 8, 128)


def _topc_sorted(Kv, Iv, c):
    L = 10
    cap = 1 << c
    for sigma in range(1, c + 1):
        for t in range(sigma - 1, -1, -1):
            Kv, Iv = _bitonic_pass(Kv, Iv, sigma, t, L)
    while L > c:
        S = 1 << L
        sh = (S // (2 * cap), 2, cap, 8, 128)
        Ka = Kv.reshape(sh)
        Ia = Iv.reshape(sh)
        lo_k, lo_i, _, _ = _ce(Ka[:, 0], Ia[:, 0], Ka[:, 1], Ia[:, 1], True)
        L -= 1
        Kv = lo_k.reshape(1 << L, 8, 128)
        Iv = lo_i.reshape(1 << L, 8, 128)
        for t in range(c - 1, -1, -1):
            Kv, Iv = _bitonic_pass(Kv, Iv, c, t, L)
    return Kv, Iv


def _sel_only_kernel(d_ref, idx_ref, *, dil):
    Kv = d_ref[...].reshape(N, 8, 128)
    Iv = lax.broadcasted_iota(jnp.int32, (N, 8, 128), 0)
    c = max(4, ((K - 1) * dil).bit_length())
    _, Iv = _topc_sorted(Kv, Iv, c)
    sel = jnp.concatenate([Iv[t * dil:t * dil + 1] for t in range(K)], axis=0)
    idx_ref[...] = sel.reshape(K, N)


def _knn_kernel(x_ref, x2_ref, idx_ref, *, dil):
    x = x_ref[0]
    x2 = x2_ref[0, 0]
    g = lax.dot_general(x, x, (((1,), (1,)), ((), ())),
                        preferred_element_type=jnp.float32)
    d = x2[:, None] + x2[None, :] - 2.0 * g
    Kv = d.reshape(N, 8, 128)
    Iv = lax.broadcasted_iota(jnp.int32, (N, 8, 128), 0)
    c = max(4, ((K - 1) * dil).bit_length())
    _, Iv = _topc_sorted(Kv, Iv, c)
    sel = jnp.concatenate([Iv[t * dil:t * dil + 1] for t in range(K)], axis=0)
    idx_ref[0] = sel.reshape(K, N)


def _knn_strided(x, dil):
    B = x.shape[0]
    C = x.shape[2]
    x2 = jnp.sum(x * x, axis=-1).reshape(B, 1, N)
    idx16 = pl.pallas_call(
        functools.partial(_knn_kernel, dil=dil),
        grid=(B,),
        in_specs=[
            pl.BlockSpec((1, N, C), lambda b: (b, 0, 0)),
            pl.BlockSpec((1, 1, N), lambda b: (b, 0, 0)),
        ],
        out_specs=pl.BlockSpec((1, K, N), lambda b: (b, 0, 0)),
        out_shape=jax.ShapeDtypeStruct((B, K, N), jnp.int32),
    )(x, x2)
    return jnp.transpose(idx16, (0, 2, 1))


_GCHUNK = 512


def _make_sc_gather(total_rows, n_idx, D):
    mesh = plsc.VectorSubcoreMesh(core_axis_name="c", subcore_axis_name="s")
    per_w = n_idx // 32
    n_chunks = per_w // _GCHUNK

    @functools.partial(
        pl.kernel,
        mesh=mesh,
        out_type=jax.ShapeDtypeStruct((n_idx, D), jnp.float32),
        scratch_types=[
            pltpu.VMEM((_GCHUNK,), jnp.int32),
            pltpu.VMEM((_GCHUNK, D), jnp.float32),
            pltpu.SemaphoreType.DMA,
        ],
    )
    def gath(table_hbm, idx_hbm, out_hbm, idx_v, rows_v, sem):
        wid = lax.axis_index("s") * 2 + lax.axis_index("c")
        base = wid * per_w
        for ci in range(n_chunks):
            off = base + ci * _GCHUNK
            pltpu.sync_copy(idx_hbm.at[pl.ds(off, _GCHUNK)], idx_v)
            pltpu.async_copy(table_hbm.at[idx_v], rows_v, sem).wait()
            pltpu.sync_copy(rows_v, out_hbm.at[pl.ds(off, _GCHUNK)])

    return gath


def _gather_neighbors(x, idx):
    B, n, C = x.shape
    Dp = 128
    xt = x.reshape(B * n, C)
    if Dp != C:
        xt = jnp.pad(xt, ((0, 0), (0, Dp - C)))
    gidx = (idx + (jnp.arange(B, dtype=idx.dtype) * n)[:, None, None]).reshape(-1)
    return _make_sc_gather(B * n, B * n * K, Dp)(xt, gidx)


_ECHUNK = 256


def _edge_conv_kernel(x_ref, xj_ref, w_ref, b_ref, o_ref, *, C, residual):
    xi = x_ref[0]
    xj = xj_ref[0][:, :C]
    xib = jnp.broadcast_to(xi[:, None, :], (_ECHUNK, K, C)).reshape(_ECHUNK * K, C)
    msg = jnp.concatenate([xib, xj - xib], axis=-1)
    h = lax.dot_general(msg, w_ref[...], (((1,), (0,)), ((), ())),
                        preferred_element_type=jnp.float32)
    if b_ref is not None:
        h = h + b_ref[0]
    h = jnp.maximum(h * BN_INV, 0.0)
    out = jnp.max(h.reshape(_ECHUNK, K, h.shape[-1]), axis=1)
    if residual:
        out = out + xi
    o_ref[0] = out


def _edge_conv(x, xj_rows, W, b, residual):
    B, n, C = x.shape
    Cout = W.shape[1]
    nchunks = n // _ECHUNK
    xj3 = xj_rows.reshape(B, n * K, 128)
    args = [x, xj3, W]
    in_specs = [
        pl.BlockSpec((1, _ECHUNK, C), lambda b_, c_: (b_, c_, 0)),
        pl.BlockSpec((1, _ECHUNK * K, 128), lambda b_, c_: (b_, c_, 0)),
        pl.BlockSpec(W.shape, lambda b_, c_: (0, 0)),
    ]
    if b is not None:
        args.append(b.reshape(1, Cout))
        in_specs.append(pl.BlockSpec((1, Cout), lambda b_, c_: (0, 0)))
        body = functools.partial(_edge_conv_kernel, C=C, residual=residual)
    else:
        body = functools.partial(
            lambda x_ref, xj_ref, w_ref, o_ref, C, residual:
            _edge_conv_kernel(x_ref, xj_ref, w_ref, None, o_ref,
                              C=C, residual=residual),
            C=C, residual=residual)
    return pl.pallas_call(
        body,
        grid=(B, nchunks),
        in_specs=in_specs,
        out_specs=pl.BlockSpec((1, _ECHUNK, Cout), lambda b_, c_: (b_, c_, 0)),
        out_shape=jax.ShapeDtypeStruct((B, n, Cout), jnp.float32),
    )(*args)


def _fusion_kernel(f_ref, wf_ref, o_ref):
    acc = jnp.dot(f_ref[...], wf_ref[...], preferred_element_type=jnp.float32)
    acc = acc * BN_INV
    o_ref[...] = jnp.where(acc >= 0.0, acc, 0.2 * acc)


def _fusion(f, Wf):
    B, n, C = f.shape
    E = Wf.shape[1]
    return pl.pallas_call(
        _fusion_kernel,
        grid=(B,),
        in_specs=[
            pl.BlockSpec((1, n, C), lambda b: (b, 0, 0)),
            pl.BlockSpec((C, E), lambda b: (0, 0)),
        ],
        out_specs=pl.BlockSpec((1, n, E), lambda b: (b, 0, 0)),
        out_shape=jax.ShapeDtypeStruct((B, n, E), jnp.float32),
    )(f, Wf)


def kernel(pts, W_head, Wb, bb, Wf):
    idx0 = _knn_strided(pts, 1)
    xj0 = _gather_neighbors(pts, idx0)
    feats = [_edge_conv(pts, xj0, W_head, None, residual=False)]
    for i in range(N_BLOCKS - 1):
        dil = i + 1
        xi = feats[-1]
        idx = _knn_strided(xi, dil)
        xj = _gather_neighbors(xi, idx)
        feats.append(_edge_conv(xi, xj, Wb[i], bb[i], residual=True))
    f = jnp.concatenate(feats, axis=-1)
    out = _fusion(f, Wf)
    return jnp.transpose(out, (0, 2, 1))


# speedup vs baseline: 6.9043x; 1.0055x over previous
import functools

import jax
import jax.numpy as jnp
import numpy as np
from jax import lax
from jax.experimental import pallas as pl
from jax.experimental.pallas import tpu as pltpu
from jax.experimental.pallas import tpu_sc as plsc

BN_INV = float(1.0 / np.sqrt(1.0 + 1e-5))
K = 16
N_BLOCKS = 14
N = 1024


def _ce(ka, ia, kb, ib, asc):
    swap = (ka > kb) | ((ka == kb) & (ia > ib))
    if not asc:
        swap = ~swap
    lo_k = jnp.where(swap, kb, ka)
    hi_k = jnp.where(swap, ka, kb)
    lo_i = jnp.where(swap, ib, ia)
    hi_i = jnp.where(swap, ia, ib)
    return lo_k, lo_i, hi_k, hi_i


def _bitonic_pass(Kv, Iv, sigma, t, L):
    S = 1 << L
    E = 1 << t
    if sigma >= L:
        shape = (1, 1, S >> (1 + t), 2, E, 8, 128)
    else:
        shape = (1 << (L - 1 - sigma), 2, 1 << (sigma - 1 - t), 2, E, 8, 128)
    Kv = Kv.reshape(shape)
    Iv = Iv.reshape(shape)
    parts_k, parts_i = [], []
    for d in range(shape[1]):
        asc = (d == 0)
        lo_k, lo_i, hi_k, hi_i = _ce(Kv[:, d, :, 0], Iv[:, d, :, 0],
                                     Kv[:, d, :, 1], Iv[:, d, :, 1], asc)
        parts_k.append(jnp.stack([lo_k, hi_k], axis=2))
        parts_i.append(jnp.stack([lo_i, hi_i], axis=2))
    Kn = parts_k[0] if len(parts_k) == 1 else jnp.stack(parts_k, axis=1)
    In = parts_i[0] if len(parts_i) == 1 else jnp.stack(parts_i, axis=1)
    return Kn.reshape(S, 8, 128), In.reshape(S, 8, 128)


def _topc_sorted(Kv, Iv, c):
    L = 10
    cap = 1 << c
    for sigma in range(1, c + 1):
        for t in range(sigma - 1, -1, -1):
            Kv, Iv = _bitonic_pass(Kv, Iv, sigma, t, L)
    while L > c:
        S = 1 << L
        sh = (S // (2 * cap), 2, cap, 8, 128)
        Ka = Kv.reshape(sh)
        Ia = Iv.reshape(sh)
        lo_k, lo_i, _, _ = _ce(Ka[:, 0], Ia[:, 0], Ka[:, 1], Ia[:, 1], True)
        L -= 1
        Kv = lo_k.reshape(1 << L, 8, 128)
        Iv = lo_i.reshape(1 << L, 8, 128)
        for t in range(c - 1, -1, -1):
            Kv, Iv = _bitonic_pass(Kv, Iv, c, t, L)
    return Kv, Iv


def _sel_only_kernel(d_ref, idx_ref, *, dil):
    Kv = d_ref[...].reshape(N, 8, 128)
    Iv = lax.broadcasted_iota(jnp.int32, (N, 8, 128), 0)
    c = max(4, ((K - 1) * dil).bit_length())
    _, Iv = _topc_sorted(Kv, Iv, c)
    sel = jnp.concatenate([Iv[t * dil:t * dil + 1] for t in range(K)], axis=0)
    idx_ref[...] = sel.reshape(K, N)


def _knn_kernel(x_ref, x2_ref, idx_ref, *, dil):
    x = x_ref[0]
    x2 = x2_ref[0, 0]
    g = lax.dot_general(x, x, (((1,), (1,)), ((), ())),
                        preferred_element_type=jnp.float32)
    d = x2[:, None] + x2[None, :] - 2.0 * g
    Kv = d.reshape(N, 8, 128)
    Iv = lax.broadcasted_iota(jnp.int32, (N, 8, 128), 0)
    c = max(4, ((K - 1) * dil).bit_length())
    _, Iv = _topc_sorted(Kv, Iv, c)
    sel = jnp.concatenate([Iv[t * dil:t * dil + 1] for t in range(K)], axis=0)
    idx_ref[0] = sel.reshape(K, N)


def _knn_strided(x, dil):
    B = x.shape[0]
    C = x.shape[2]
    x2 = jnp.sum(x * x, axis=-1).reshape(B, 1, N)
    idx16 = pl.pallas_call(
        functools.partial(_knn_kernel, dil=dil),
        grid=(B,),
        in_specs=[
            pl.BlockSpec((1, N, C), lambda b: (b, 0, 0)),
            pl.BlockSpec((1, 1, N), lambda b: (b, 0, 0)),
        ],
        out_specs=pl.BlockSpec((1, K, N), lambda b: (b, 0, 0)),
        out_shape=jax.ShapeDtypeStruct((B, K, N), jnp.int32),
    )(x, x2)
    return jnp.transpose(idx16, (0, 2, 1))


_GCHUNK = 128


def _make_sc_gather(total_rows, n_idx, D):
    mesh = plsc.VectorSubcoreMesh(core_axis_name="c", subcore_axis_name="s")
    per_w = n_idx // 32
    n_chunks = per_w // _GCHUNK

    @functools.partial(
        pl.kernel,
        mesh=mesh,
        out_type=jax.ShapeDtypeStruct((n_idx, D), jnp.float32),
        scratch_types=[
            pltpu.VMEM((n_chunks, _GCHUNK), jnp.int32),
            pltpu.VMEM((2, _GCHUNK, D), jnp.float32),
            pltpu.SemaphoreType.DMA,
            pltpu.SemaphoreType.DMA,
        ],
    )
    def gath(table_hbm, idx_hbm, out_hbm, idx_v, rows_v, sem0, sem1):
        wid = lax.axis_index("s") * 2 + lax.axis_index("c")
        base = wid * per_w
        sems = (sem0, sem1)
        pltpu.sync_copy(idx_hbm.at[wid], idx_v)
        copies = [None] * n_chunks
        copies[0] = pltpu.async_copy(table_hbm.at[idx_v.at[0]],
                                     rows_v.at[0], sems[0])
        for ci in range(n_chunks):
            b = ci & 1
            nxt = ci + 1
            if nxt < n_chunks:
                copies[nxt] = pltpu.async_copy(table_hbm.at[idx_v.at[nxt]],
                                               rows_v.at[nxt & 1], sems[nxt & 1])
            copies[ci].wait()
            pltpu.sync_copy(rows_v.at[b],
                            out_hbm.at[pl.ds(base + ci * _GCHUNK, _GCHUNK)])

    return gath


def _gather_neighbors(x, idx):
    B, n, C = x.shape
    Dp = 128
    xt = x.reshape(B * n, C)
    if Dp != C:
        xt = jnp.pad(xt, ((0, 0), (0, Dp - C)))
    gidx = (idx + (jnp.arange(B, dtype=idx.dtype) * n)[:, None, None]).reshape(
        32, (B * n * K) // (32 * _GCHUNK), _GCHUNK)
    return _make_sc_gather(B * n, B * n * K, Dp)(xt, gidx)


_ECHUNK = 256


def _edge_conv_kernel(x_ref, xj_ref, w_ref, b_ref, o_ref, *, C, residual):
    xi = x_ref[0]
    xj = xj_ref[0][:, :C]
    xib = jnp.broadcast_to(xi[:, None, :], (_ECHUNK, K, C)).reshape(_ECHUNK * K, C)
    msg = jnp.concatenate([xib, xj - xib], axis=-1)
    h = lax.dot_general(msg, w_ref[...], (((1,), (0,)), ((), ())),
                        preferred_element_type=jnp.float32)
    if b_ref is not None:
        h = h + b_ref[0]
    h = jnp.maximum(h * BN_INV, 0.0)
    out = jnp.max(h.reshape(_ECHUNK, K, h.shape[-1]), axis=1)
    if residual:
        out = out + xi
    o_ref[0] = out


def _edge_conv(x, xj_rows, W, b, residual):
    B, n, C = x.shape
    Cout = W.shape[1]
    nchunks = n // _ECHUNK
    xj3 = xj_rows.reshape(B, n * K, 128)
    args = [x, xj3, W]
    in_specs = [
        pl.BlockSpec((1, _ECHUNK, C), lambda b_, c_: (b_, c_, 0)),
        pl.BlockSpec((1, _ECHUNK * K, 128), lambda b_, c_: (b_, c_, 0)),
        pl.BlockSpec(W.shape, lambda b_, c_: (0, 0)),
    ]
    if b is not None:
        args.append(b.reshape(1, Cout))
        in_specs.append(pl.BlockSpec((1, Cout), lambda b_, c_: (0, 0)))
        body = functools.partial(_edge_conv_kernel, C=C, residual=residual)
    else:
        body = functools.partial(
            lambda x_ref, xj_ref, w_ref, o_ref, C, residual:
            _edge_conv_kernel(x_ref, xj_ref, w_ref, None, o_ref,
                              C=C, residual=residual),
            C=C, residual=residual)
    return pl.pallas_call(
        body,
        grid=(B, nchunks),
        in_specs=in_specs,
        out_specs=pl.BlockSpec((1, _ECHUNK, Cout), lambda b_, c_: (b_, c_, 0)),
        out_shape=jax.ShapeDtypeStruct((B, n, Cout), jnp.float32),
    )(*args)


def _fusion_kernel(f_ref, wf_ref, o_ref):
    acc = jnp.dot(f_ref[...], wf_ref[...], preferred_element_type=jnp.float32)
    acc = acc * BN_INV
    o_ref[...] = jnp.where(acc >= 0.0, acc, 0.2 * acc)


def _fusion(f, Wf):
    B, n, C = f.shape
    E = Wf.shape[1]
    return pl.pallas_call(
        _fusion_kernel,
        grid=(B,),
        in_specs=[
            pl.BlockSpec((1, n, C), lambda b: (b, 0, 0)),
            pl.BlockSpec((C, E), lambda b: (0, 0)),
        ],
        out_specs=pl.BlockSpec((1, n, E), lambda b: (b, 0, 0)),
        out_shape=jax.ShapeDtypeStruct((B, n, E), jnp.float32),
    )(f, Wf)


def kernel(pts, W_head, Wb, bb, Wf):
    idx0 = _knn_strided(pts, 1)
    xj0 = _gather_neighbors(pts, idx0)
    feats = [_edge_conv(pts, xj0, W_head, None, residual=False)]
    for i in range(N_BLOCKS - 1):
        dil = i + 1
        xi = feats[-1]
        idx = _knn_strided(xi, dil)
        xj = _gather_neighbors(xi, idx)
        feats.append(_edge_conv(xi, xj, Wb[i], bb[i], residual=True))
    f = jnp.concatenate(feats, axis=-1)
    out = _fusion(f, Wf)
    return jnp.transpose(out, (0, 2, 1))
